# baseline (speedup 1.0000x reference)
import jax
import jax.numpy as jnp
from jax import lax
from jax.experimental import pallas as pl
from jax.experimental.pallas import tpu as pltpu
from jax.experimental.pallas import tpu_sc as plsc

_SHIFT = 1000
_T = 8192
_ROWS = 4096
_NC = 2
_NS = 16
_NW = _NC * _NS
_RPW = _ROWS // _NW
_RCHUNK = 4
_NCHUNK = _RPW // _RCHUNK
_PADT = 9216
_LANES = 16


def _roll_body(x_hbm, out_hbm, buf, sem_in, sem_out):
    wid = lax.axis_index("s") * _NC + lax.axis_index("c")
    r0 = wid * _RPW

    def chunk(i, carry):
        rows = r0 + i * _RCHUNK
        pltpu.async_copy(
            x_hbm.at[pl.ds(rows, _RCHUNK), :],
            buf.at[:, pl.ds(_SHIFT, _T)],
            sem_in,
        ).wait()
        for r in range(_RCHUNK):
            for j in range(62):
                buf[r, pl.ds(_LANES * j, _LANES)] = buf[
                    r, pl.ds(_T + _LANES * j, _LANES)
                ]
            buf[r, pl.ds(984, _LANES)] = buf[r, pl.ds(9176, _LANES)]
        pltpu.async_copy(
            buf.at[:, pl.ds(0, _T)],
            out_hbm.at[pl.ds(rows, _RCHUNK), :],
            sem_out,
        ).wait()
        return carry

    lax.fori_loop(0, _NCHUNK, chunk, 0)


def kernel(x):
    b, s, t = x.shape
    x2 = x.reshape(b * s, t)
    mesh = plsc.VectorSubcoreMesh(core_axis_name="c", subcore_axis_name="s")
    out = pl.kernel(
        _roll_body,
        out_type=jax.ShapeDtypeStruct((b * s, t), x.dtype),
        mesh=mesh,
        scratch_types=[
            pltpu.VMEM((_RCHUNK, _PADT), jnp.float32),
            pltpu.SemaphoreType.DMA,
            pltpu.SemaphoreType.DMA,
        ],
        compiler_params=pltpu.CompilerParams(use_tc_tiling_on_sc=False),
    )(x2)
    return out.reshape(b, s, t)


# speedup vs baseline: 1.0626x; 1.0626x over previous
import jax
import jax.numpy as jnp
from jax import lax
from jax.experimental import pallas as pl
from jax.experimental.pallas import tpu as pltpu
from jax.experimental.pallas import tpu_sc as plsc

_SHIFT = 1000
_T = 8192
_ROWS = 4096
_NC = 2
_NS = 16
_NW = _NC * _NS
_RPW = _ROWS // _NW
_RCHUNK = 2
_NCHUNK = _RPW // _RCHUNK
_NBUF = 4
_PADT = 9216
_LANES = 16


def _roll_body(x_hbm, out_hbm, b0, b1, b2, b3,
               l0, l1, l2, l3, s0, s1, s2, s3):
    bufs = (b0, b1, b2, b3)
    lsems = (l0, l1, l2, l3)
    ssems = (s0, s1, s2, s3)
    wid = lax.axis_index("s") * _NC + lax.axis_index("c")
    r0 = wid * _RPW

    def load(i, slot):
        return pltpu.make_async_copy(
            x_hbm.at[pl.ds(r0 + i * _RCHUNK, _RCHUNK), :],
            bufs[slot].at[:, pl.ds(_SHIFT, _T)],
            lsems[slot],
        )

    def store(i, slot):
        return pltpu.make_async_copy(
            bufs[slot].at[:, pl.ds(0, _T)],
            out_hbm.at[pl.ds(r0 + i * _RCHUNK, _RCHUNK), :],
            ssems[slot],
        )

    def fix(slot):
        buf = bufs[slot]
        for r in range(_RCHUNK):
            for j in range(62):
                buf[r, pl.ds(_LANES * j, _LANES)] = buf[
                    r, pl.ds(_T + _LANES * j, _LANES)
                ]
            buf[r, pl.ds(984, _LANES)] = buf[r, pl.ds(9176, _LANES)]

    for k in range(_NBUF - 1):
        load(k, k).start()

    def chunk_step(i, slot, prefetch, store_wait):
        load(i, slot).wait()
        fix(slot)
        store(i, slot).start()
        if prefetch:
            k = i + _NBUF - 1
            kslot = (slot + _NBUF - 1) % _NBUF
            if store_wait:
                store(k - _NBUF, kslot).wait()
            load(k, kslot).start()

    chunk_step(0, 0, True, False)
    for slot in range(1, _NBUF):
        chunk_step(slot, slot, True, True)

    def group(g, carry):
        for slot in range(_NBUF):
            chunk_step(g * _NBUF + slot, slot, True, True)
        return carry

    lax.fori_loop(1, _NCHUNK // _NBUF - 1, group, 0)

    base = _NCHUNK - _NBUF
    chunk_step(base, 0, True, True)
    for slot in range(1, _NBUF):
        chunk_step(base + slot, slot, False, False)

    for slot in range(_NBUF):
        store(base + slot, slot).wait()


def kernel(x):
    b, s, t = x.shape
    x2 = x.reshape(b * s, t)
    mesh = plsc.VectorSubcoreMesh(core_axis_name="c", subcore_axis_name="s")
    out = pl.kernel(
        _roll_body,
        out_type=jax.ShapeDtypeStruct((b * s, t), x.dtype),
        mesh=mesh,
        scratch_types=(
            [pltpu.VMEM((_RCHUNK, _PADT), jnp.float32) for _ in range(_NBUF)]
            + [pltpu.SemaphoreType.DMA for _ in range(2 * _NBUF)]
        ),
        compiler_params=pltpu.CompilerParams(use_tc_tiling_on_sc=False),
    )(x2)
    return out.reshape(b, s, t)
